# Initial kernel scaffold; baseline (speedup 1.0000x reference)
import jax
import jax.numpy as jnp
from jax.experimental import pallas as pl


def kernel(node_pos, node_irreps, edge_dis_embedding, src_embed, tgt_embed, Wq, Wk, alpha_dot, W1a, b1a, W2a, b2a, W3a, b3a, W1b, b1b, W2b, b2b, W3b, b3b, Wproj, atomic_numbers, edge_index):
    raise NotImplementedError("write your pallas kernel here")


# speedup vs baseline: 1.7937x; 1.7937x over previous
import functools
import math

import jax
import jax.numpy as jnp
import numpy as np
from jax.experimental import pallas as pl

N = 10000
E = 320000
L2 = 9
C = 256
H = 8
DH = 32
AW = 32
NE = 128

_BE = 2000
_BM = 512
_BN = 1000
_BP = 2000


def _qk_kernel(x_ref, wq_ref, wk_ref, q_ref, k_ref):
    x = x_ref[...]
    q_ref[...] = jnp.dot(x, wq_ref[...], preferred_element_type=jnp.float32)
    k_ref[...] = jnp.dot(x, wk_ref[...], preferred_element_type=jnp.float32)


def _edge_kernel(dis_ref, embs_ref, embt_ref, scal_ref, qd_ref, ks_ref,
                 w1ad_ref, w1ae_ref, w1as_ref, b1a_ref, w2a_ref, b2a_ref,
                 w3a_ref, b3a_ref,
                 w1bd_ref, w1be_ref, w1bs_ref, b1b_ref, w2b_ref, b2b_ref,
                 w3b_ref, b3b_ref, adot_ref, g_ref,
                 alpha_ref, vw_ref):
    dis = dis_ref[...]
    emb = embs_ref[...] + embt_ref[...]
    scal = scal_ref[...]

    def mlp(w1d, w1e, w1s, b1, w2, b2, w3, b3):
        h = (jnp.dot(dis, w1d, preferred_element_type=jnp.float32)
             + jnp.dot(emb, w1e, preferred_element_type=jnp.float32)
             + jnp.dot(scal, w1s, preferred_element_type=jnp.float32)
             + b1)
        h = h * jax.nn.sigmoid(h)
        h = jnp.dot(h, w2, preferred_element_type=jnp.float32) + b2
        h = h * jax.nn.sigmoid(h)
        return jnp.dot(h, w3, preferred_element_type=jnp.float32) + b3

    alpha_bias = mlp(w1ad_ref[...], w1ae_ref[...], w1as_ref[...], b1a_ref[...],
                     w2a_ref[...], b2a_ref[...], w3a_ref[...], b3a_ref[...])
    vw = mlp(w1bd_ref[...], w1be_ref[...], w1bs_ref[...], b1b_ref[...],
             w2b_ref[...], b2b_ref[...], w3b_ref[...], b3b_ref[...])

    pre = qd_ref[...] * ks_ref[...]
    slr = 0.8 * pre * jax.nn.sigmoid(pre) + 0.2 * pre
    logits = jnp.dot(slr * adot_ref[...], g_ref[...],
                     preferred_element_type=jnp.float32)
    alpha_ref[...] = logits * (1.0 / math.sqrt(DH)) + alpha_bias
    vw_ref[...] = vw


def _msg_kernel(irr_ref, vw_ref, ex_ref, den_ref, gt_ref, out_ref):
    attn = ex_ref[...] / (den_ref[...] + 1e-9)
    w = jnp.dot(attn, gt_ref[...], preferred_element_type=jnp.float32)
    w = w * vw_ref[...]
    out_ref[...] = irr_ref[...] * w[:, None, :]


def _proj_kernel(x_ref, w_ref, o_ref):
    o_ref[...] = jnp.dot(x_ref[...], w_ref[...],
                         preferred_element_type=jnp.float32)


def _full(shape):
    return pl.BlockSpec(shape, lambda i: tuple(0 for _ in shape))


@jax.jit
def kernel(node_pos, node_irreps, edge_dis_embedding, src_embed, tgt_embed,
           Wq, Wk, alpha_dot, W1a, b1a, W2a, b2a, W3a, b3a,
           W1b, b1b, W2b, b2b, W3b, b3b, Wproj, atomic_numbers, edge_index):
    src = edge_index[0]
    dst = edge_index[1]

    x = node_irreps.reshape(N, L2 * C)
    wq = Wq.reshape(L2 * C, H * DH)
    wk = Wk.reshape(L2 * C, H * DH)
    q, k = pl.pallas_call(
        _qk_kernel,
        grid=(N // _BN,),
        in_specs=[
            pl.BlockSpec((_BN, L2 * C), lambda i: (i, 0)),
            _full((L2 * C, H * DH)),
            _full((L2 * C, H * DH)),
        ],
        out_specs=[
            pl.BlockSpec((_BN, H * DH), lambda i: (i, 0)),
            pl.BlockSpec((_BN, H * DH), lambda i: (i, 0)),
        ],
        out_shape=[
            jax.ShapeDtypeStruct((N, H * DH), jnp.float32),
            jax.ShapeDtypeStruct((N, H * DH), jnp.float32),
        ],
    )(x, wq, wk)

    emb_s = jnp.take(src_embed, atomic_numbers, axis=0)
    emb_t = jnp.take(tgt_embed, atomic_numbers, axis=0)
    scal = node_irreps[:, 0, :]
    embs_e = jnp.take(emb_s, src, axis=0)
    embt_e = jnp.take(emb_t, dst, axis=0)
    scal_e = jnp.take(scal, src, axis=0)
    qd = jnp.take(q, dst, axis=0)
    ks = jnp.take(k, src, axis=0)

    g = (jnp.arange(H * DH)[:, None] // DH
         == jnp.arange(H)[None, :]).astype(jnp.float32)
    gt = g.T
    adot = alpha_dot.reshape(1, H * DH)

    row = lambda b: b.reshape(1, -1)
    weights = (W1a[:AW], W1a[AW:AW + NE], W1a[AW + NE:], row(b1a), W2a,
               row(b2a), W3a, row(b3a),
               W1b[:AW], W1b[AW:AW + NE], W1b[AW + NE:], row(b1b), W2b,
               row(b2b), W3b, row(b3b), adot, g)
    wspecs = [_full(w.shape) for w in weights]

    eb = lambda d: pl.BlockSpec((_BE, d), lambda i: (i, 0))
    alpha, vw = pl.pallas_call(
        _edge_kernel,
        grid=(E // _BE,),
        in_specs=[eb(AW), eb(NE), eb(NE), eb(C), eb(H * DH), eb(H * DH)]
        + wspecs,
        out_specs=[eb(H), eb(C)],
        out_shape=[
            jax.ShapeDtypeStruct((E, H), jnp.float32),
            jax.ShapeDtypeStruct((E, C), jnp.float32),
        ],
    )(edge_dis_embedding, embs_e, embt_e, scal_e, qd, ks, *weights)

    amax = jax.ops.segment_max(alpha, dst, num_segments=N)
    amax = jnp.where(jnp.isfinite(amax), amax, 0.0)
    ex = jnp.exp(alpha - amax[dst])
    denom = jax.ops.segment_sum(ex, dst, num_segments=N)
    den_e = jnp.take(denom, dst, axis=0)

    irr_e = jnp.take(node_irreps, src, axis=0)
    mb = lambda d: pl.BlockSpec((_BM, d), lambda i: (i, 0))
    msg = pl.pallas_call(
        _msg_kernel,
        grid=(E // _BM,),
        in_specs=[
            pl.BlockSpec((_BM, L2, C), lambda i: (i, 0, 0)),
            mb(C), mb(H), mb(H), _full((H, H * DH)),
        ],
        out_specs=pl.BlockSpec((_BM, L2, C), lambda i: (i, 0, 0)),
        out_shape=jax.ShapeDtypeStruct((E, L2, C), jnp.float32),
    )(irr_e, vw, ex, den_e, gt)

    agg = jax.ops.segment_sum(msg.reshape(E, L2 * C), dst, num_segments=N)
    out = pl.pallas_call(
        _proj_kernel,
        grid=(N * L2 // _BP,),
        in_specs=[pl.BlockSpec((_BP, C), lambda i: (i, 0)), _full((C, C))],
        out_specs=pl.BlockSpec((_BP, C), lambda i: (i, 0)),
        out_shape=jax.ShapeDtypeStruct((N * L2, C), jnp.float32),
    )(agg.reshape(N * L2, C), Wproj)
    return out.reshape(N, L2, C)
